# Initial kernel scaffold; baseline (speedup 1.0000x reference)
import jax
import jax.numpy as jnp
from jax.experimental import pallas as pl


def kernel(x, emb):
    raise NotImplementedError("write your pallas kernel here")


# speedup vs baseline: 1.6655x; 1.6655x over previous
import jax
import jax.numpy as jnp
from jax.experimental import pallas as pl


_SEQ_TILE = 1024


def _add_kernel(x_ref, e_ref, o_ref):
    o_ref[...] = x_ref[...] + e_ref[...]


def kernel(x, emb):
    B, S, D = x.shape
    ts = _SEQ_TILE
    grid = (S // ts, B)
    return pl.pallas_call(
        _add_kernel,
        grid=grid,
        in_specs=[
            pl.BlockSpec((1, ts, D), lambda s, b: (b, s, 0)),
            pl.BlockSpec((ts, D), lambda s, b: (s, 0)),
        ],
        out_specs=pl.BlockSpec((1, ts, D), lambda s, b: (b, s, 0)),
        out_shape=jax.ShapeDtypeStruct(x.shape, x.dtype),
    )(x, emb)


# speedup vs baseline: 1.7379x; 1.0434x over previous
import jax
import jax.numpy as jnp
from jax.experimental import pallas as pl


_SEQ_TILE = 2048


def _add_kernel(x_ref, e_ref, o_ref):
    o_ref[...] = x_ref[...] + e_ref[...]


def kernel(x, emb):
    B, S, D = x.shape
    ts = _SEQ_TILE
    grid = (S // ts, B)
    return pl.pallas_call(
        _add_kernel,
        grid=grid,
        in_specs=[
            pl.BlockSpec((1, ts, D), lambda s, b: (b, s, 0)),
            pl.BlockSpec((ts, D), lambda s, b: (s, 0)),
        ],
        out_specs=pl.BlockSpec((1, ts, D), lambda s, b: (b, s, 0)),
        out_shape=jax.ShapeDtypeStruct(x.shape, x.dtype),
    )(x, emb)
